# baseline (speedup 1.0000x reference)
import functools

import jax
import jax.numpy as jnp
from jax import lax
from jax.experimental import pallas as pl
from jax.experimental.pallas import tpu as pltpu
from jax.experimental.pallas import tpu_sc as plsc

N = 384
C = 96
B = 4
PIX = N * N
ROWS = B * PIX

NC = 2
NS = 16
NW = NC * NS
ROWS_PER_W = ROWS // NW
CHUNK = 128
NCHUNKS = ROWS_PER_W // CHUNK
VEC = 16


def _arnold_gather_kernel(src_hbm, out_hbm, idx_v, buf_v, sem):
    wid = lax.axis_index("s") * NC + lax.axis_index("c")
    base = wid * ROWS_PER_W
    img_base = (base // PIX) * PIX

    def chunk_body(ci, _):
        start = base + ci * CHUNK
        for t in range(CHUNK // VEC):
            g = start + t * VEC + lax.iota(jnp.int32, VEC)
            l = g - img_base
            a = lax.div(l, N)
            b = lax.rem(l, N)
            r = lax.rem(41 * a + 29 * b, N)
            c = lax.rem(58 * a + 41 * b, N)
            idx_v[pl.ds(t * VEC, VEC)] = img_base + r * N + c
        pltpu.async_copy(src_hbm.at[idx_v], buf_v, sem).wait()
        pltpu.sync_copy(buf_v, out_hbm.at[pl.ds(start, CHUNK)])
        return _

    lax.fori_loop(0, NCHUNKS, chunk_body, 0, unroll=False)


@jax.jit
def _arnold(src):
    mesh = plsc.VectorSubcoreMesh(core_axis_name="c", subcore_axis_name="s")
    return pl.kernel(
        _arnold_gather_kernel,
        out_type=jax.ShapeDtypeStruct((ROWS, C), jnp.float32),
        mesh=mesh,
        compiler_params=pltpu.CompilerParams(use_tc_tiling_on_sc=False),
        scratch_types=[
            pltpu.VMEM((CHUNK,), jnp.int32),
            pltpu.VMEM((CHUNK, C), jnp.float32),
            pltpu.SemaphoreType.DMA,
        ],
    )(src)


def kernel(inputs):
    src = inputs.reshape(ROWS, C)
    out = _arnold(src)
    return out.reshape(B, N, N, C)


# speedup vs baseline: 1.2659x; 1.2659x over previous
import jax
import jax.numpy as jnp
from jax import lax
from jax.experimental import pallas as pl
from jax.experimental.pallas import tpu as pltpu
from jax.experimental.pallas import tpu_sc as plsc

N = 384
C = 96
CP = 128
B = 4
PIX = N * N
ROWS = B * PIX

NC = 2
NS = 16
NW = NC * NS
ROWS_PER_W = ROWS // NW
CHUNK = 128
NCHUNKS = ROWS_PER_W // CHUNK
VEC = 16


def _arnold_gather_kernel(src_hbm, out_hbm, idx_v, buf_v, sem):
    out2 = out_hbm
    wid = lax.axis_index("s") * NC + lax.axis_index("c")
    base = wid * ROWS_PER_W
    img_base = (base // PIX) * PIX

    def chunk_body(ci, _):
        start = base + ci * CHUNK
        for t in range(CHUNK // VEC):
            g = start + t * VEC + lax.iota(jnp.int32, VEC)
            l = g - img_base
            a = lax.div(l, N)
            b = lax.rem(l, N)
            r = lax.rem(41 * a + 29 * b, N)
            c = lax.rem(58 * a + 41 * b, N)
            idx_v[pl.ds(t * VEC, VEC)] = img_base + r * N + c
        pltpu.async_copy(src_hbm.at[idx_v], buf_v, sem).wait()
        pltpu.sync_copy(buf_v, out2.at[pl.ds(start, CHUNK)])
        return _

    lax.fori_loop(0, NCHUNKS, chunk_body, 0, unroll=False)


@jax.jit
def _arnold(src):
    mesh = plsc.VectorSubcoreMesh(core_axis_name="c", subcore_axis_name="s")
    return pl.kernel(
        _arnold_gather_kernel,
        out_type=jax.ShapeDtypeStruct((ROWS, CP), jnp.float32),
        mesh=mesh,
        scratch_types=[
            pltpu.VMEM((CHUNK,), jnp.int32),
            pltpu.VMEM((CHUNK, CP), jnp.float32),
            pltpu.SemaphoreType.DMA,
        ],
    )(src)


def kernel(inputs):
    x = jnp.pad(inputs, ((0, 0), (0, 0), (0, 0), (0, CP - C)))
    src = x.reshape(ROWS, CP)
    out = _arnold(src)
    return out.reshape(B, N, N, CP)[..., :C]


# speedup vs baseline: 1.9067x; 1.5061x over previous
import jax
import jax.numpy as jnp
from jax import lax
from jax.experimental import pallas as pl
from jax.experimental.pallas import tpu as pltpu
from jax.experimental.pallas import tpu_sc as plsc

N = 384
C = 96
CP = 128
B = 4
PIX = N * N
ROWS = B * PIX

NC = 2
NS = 16
NW = NC * NS
ROWS_PER_W = ROWS // NW
CHUNK = 128
NCHUNKS = ROWS_PER_W // CHUNK
VEC = 16
NBUF = 3


def _arnold_gather_kernel(src_hbm, out_hbm, idx_v, buf_v,
                          sg0, sg1, sg2, sw0, sw1, sw2):
    sem_g = (sg0, sg1, sg2)
    sem_w = (sw0, sw1, sw2)
    wid = lax.axis_index("s") * NC + lax.axis_index("c")
    base = wid * ROWS_PER_W
    img_base = (base // PIX) * PIX

    def fill_idx(slot, start):
        row = idx_v.at[slot]

        def t_body(t, carry):
            g = start + t * VEC + lax.iota(jnp.int32, VEC)
            l = g - img_base
            a = lax.div(l, N)
            b = lax.rem(l, N)
            r = lax.rem(41 * a + 29 * b, N)
            c = lax.rem(58 * a + 41 * b, N)
            row[pl.ds(t * VEC, VEC)] = img_base + r * N + c
            return carry

        lax.fori_loop(0, CHUNK // VEC, t_body, 0, unroll=False)

    def fire_gather(slot, start):
        fill_idx(slot, start)
        pltpu.async_copy(src_hbm.at[idx_v.at[slot]], buf_v.at[slot],
                         sem_g[slot])

    def wait_gather(slot):
        pltpu.make_async_copy(src_hbm.at[idx_v.at[slot]], buf_v.at[slot],
                              sem_g[slot]).wait()

    def fire_write(slot, start):
        pltpu.async_copy(buf_v.at[slot], out_hbm.at[pl.ds(start, CHUNK)],
                         sem_w[slot])

    def wait_write(slot):
        pltpu.make_async_copy(buf_v.at[slot], out_hbm.at[pl.ds(0, CHUNK)],
                              sem_w[slot]).wait()

    for slot in range(NBUF):
        fire_gather(slot, base + slot * CHUNK)

    def outer_body(o, carry):
        c0 = o * NBUF
        for slot in range(NBUF):
            ci = c0 + slot
            start = base + ci * CHUNK
            prev = (slot - 1) % NBUF
            cprev = ci - 1 + NBUF

            @pl.when(jnp.logical_and(cprev >= NBUF, cprev < NCHUNKS))
            def _():
                wait_write(prev)
                fire_gather(prev, base + cprev * CHUNK)

            wait_gather(slot)
            fire_write(slot, start)
        return carry

    lax.fori_loop(0, NCHUNKS // NBUF, outer_body, 0, unroll=False)

    for slot in range(NBUF):
        wait_write(slot)


@jax.jit
def _arnold(src):
    mesh = plsc.VectorSubcoreMesh(core_axis_name="c", subcore_axis_name="s")
    return pl.kernel(
        _arnold_gather_kernel,
        out_type=jax.ShapeDtypeStruct((ROWS, CP), jnp.float32),
        mesh=mesh,
        scratch_types=[
            pltpu.VMEM((NBUF, CHUNK), jnp.int32),
            pltpu.VMEM((NBUF, CHUNK, CP), jnp.float32),
            pltpu.SemaphoreType.DMA,
            pltpu.SemaphoreType.DMA,
            pltpu.SemaphoreType.DMA,
            pltpu.SemaphoreType.DMA,
            pltpu.SemaphoreType.DMA,
            pltpu.SemaphoreType.DMA,
        ],
    )(src)


def kernel(inputs):
    x = jnp.pad(inputs, ((0, 0), (0, 0), (0, 0), (0, CP - C)))
    src = x.reshape(ROWS, CP)
    out = _arnold(src)
    return out.reshape(B, N, N, CP)[..., :C]
